# baseline (speedup 1.0000x reference)
import functools

import jax
import jax.numpy as jnp
import numpy as np
from jax import lax
from jax.experimental import pallas as pl
from jax.experimental.pallas import tpu as pltpu
from jax.experimental.pallas import tpu_sc as plsc

_NC, _NS, _LANES = 2, 16, 16
_NW = _NC * _NS
_CH = 128


def _gather_pair_add(dst_i, src_i, A, B, width):
    E = dst_i.shape[0]
    per_w = E // _NW
    n_chunks = per_w // _CH
    nvec = width // _LANES
    mesh = plsc.VectorSubcoreMesh(core_axis_name="c", subcore_axis_name="s")

    @functools.partial(
        pl.kernel,
        mesh=mesh,
        out_type=jax.ShapeDtypeStruct((E, width), jnp.float32),
        scratch_types=[
            pltpu.VMEM((_CH,), jnp.int32),
            pltpu.VMEM((_CH,), jnp.int32),
            pltpu.VMEM((_CH, width), jnp.float32),
            pltpu.VMEM((_CH, width), jnp.float32),
            pltpu.SemaphoreType.DMA,
            pltpu.SemaphoreType.DMA,
        ],
    )
    def k(dst_hbm, src_hbm, a_hbm, b_hbm, out_hbm, idx_d, idx_s, arows, brows, sem_a, sem_b):
        wid = lax.axis_index("s") * _NC + lax.axis_index("c")
        base = wid * per_w

        def chunk(ci, _):
            off = base + ci * _CH
            pltpu.sync_copy(dst_hbm.at[pl.ds(off, _CH)], idx_d)
            pltpu.sync_copy(src_hbm.at[pl.ds(off, _CH)], idx_s)
            ca = pltpu.async_copy(a_hbm.at[idx_d], arows, sem_a)
            cb = pltpu.async_copy(b_hbm.at[idx_s], brows, sem_b)
            ca.wait()
            cb.wait()

            def edge(e, _):
                for j in range(nvec):
                    sl = pl.ds(j * _LANES, _LANES)
                    arows[e, sl] = arows[e, sl] + brows[e, sl]
                return 0

            lax.fori_loop(0, _CH, edge, 0, unroll=2)
            pltpu.sync_copy(arows, out_hbm.at[pl.ds(off, _CH)])
            return 0

        lax.fori_loop(0, n_chunks, chunk, 0)

    return k(dst_i, src_i, A, B)


def _gather_rows(idx, table):
    E = idx.shape[0]
    per_w = E // _NW
    n_chunks = per_w // _CH
    mesh = plsc.VectorSubcoreMesh(core_axis_name="c", subcore_axis_name="s")

    @functools.partial(
        pl.kernel,
        mesh=mesh,
        out_type=jax.ShapeDtypeStruct((E, 128), jnp.float32),
        scratch_types=[
            pltpu.VMEM((_CH,), jnp.int32),
            pltpu.VMEM((_CH,), jnp.int32),
            pltpu.VMEM((_CH, 128), jnp.float32),
            pltpu.VMEM((_CH, 128), jnp.float32),
            pltpu.SemaphoreType.DMA,
            pltpu.SemaphoreType.DMA,
        ],
    )
    def k(idx_hbm, t_hbm, out_hbm, iv0, iv1, b0, b1, s0, s1):
        wid = lax.axis_index("s") * _NC + lax.axis_index("c")
        base = wid * per_w

        def chunk(ci, _):
            off = base + ci * (2 * _CH)
            pltpu.sync_copy(idx_hbm.at[pl.ds(off, _CH)], iv0)
            pltpu.sync_copy(idx_hbm.at[pl.ds(off + _CH, _CH)], iv1)
            c0 = pltpu.async_copy(t_hbm.at[iv0], b0, s0)
            c1 = pltpu.async_copy(t_hbm.at[iv1], b1, s1)
            c0.wait()
            pltpu.sync_copy(b0, out_hbm.at[pl.ds(off, _CH)])
            c1.wait()
            pltpu.sync_copy(b1, out_hbm.at[pl.ds(off + _CH, _CH)])
            return 0

        lax.fori_loop(0, n_chunks // 2, chunk, 0)

    return k(idx, table)


_NB = 2
_SROWS = 10240


def _sage_agg(src_i, dst_i, attr, T):
    E = src_i.shape[0]
    N = T.shape[0] // 4
    per_tile = E // _NS
    n_steps = per_tile // (_NB * _CH)
    mesh = plsc.VectorSubcoreMesh(core_axis_name="c", subcore_axis_name="s")

    @functools.partial(
        pl.kernel,
        mesh=mesh,
        out_type=jax.ShapeDtypeStruct((2, _SROWS, 128), jnp.float32),
        scratch_types=(
            [pltpu.VMEM((_NB * _CH,), jnp.int32)] * 3
            + [pltpu.VMEM((_CH,), jnp.int32)] * (2 * _NB)
            + [pltpu.VMEM((_CH, 128), jnp.float32)] * _NB
            + [pltpu.VMEM_SHARED((_SROWS, 128), jnp.float32),
               pltpu.SemaphoreType.DMA]
        ),
    )
    def k(src_hbm, dst_hbm, attr_hbm, t_hbm, out_hbm,
          sv, dv, av, gi0, gi1, ri0, ri1,
          gb0, gb1, stable, sem):
        c = lax.axis_index("c")
        s = lax.axis_index("s")
        gis = [gi0, gi1]
        ris = [ri0, ri1]
        gbs = [gb0, gb1]

        def zrow(r, _):
            for j in range(8):
                gb0[r, pl.ds(j * _LANES, _LANES)] = jnp.zeros((_LANES,), jnp.float32)
            return 0

        lax.fori_loop(0, _CH, zrow, 0)
        slab = _SROWS // _NS

        def zslab(kk, _):
            stable_blk = stable.at[pl.ds(s * slab + kk * _CH, _CH)]
            pltpu.sync_copy(gb0, stable_blk)
            return 0

        lax.fori_loop(0, slab // _CH, zslab, 0)
        plsc.subcore_barrier()

        base = s * per_tile
        c2n = c * (2 * N)

        def step(i, _):
            off = base + i * (_NB * _CH)
            pltpu.sync_copy(src_hbm.at[pl.ds(off, _NB * _CH)], sv)
            pltpu.sync_copy(dst_hbm.at[pl.ds(off, _NB * _CH)], dv)
            pltpu.sync_copy(attr_hbm.at[pl.ds(off, _NB * _CH)], av)
            for b in range(_NB):
                for j in range(_CH // _LANES):
                    sl = pl.ds(b * _CH + j * _LANES, _LANES)
                    sl_o = pl.ds(j * _LANES, _LANES)
                    s16 = sv[sl]
                    d16 = dv[sl]
                    a16 = av[sl]
                    valid = ((a16 <= 1) & (d16 < 38400)
                             & ((d16 & 255) < 64))
                    srow = ((d16 >> 8) << 6) | (d16 & 63)
                    ris[b][sl_o] = jnp.where(valid, srow, 9600)
                    gis[b][sl_o] = c2n + s16 + jnp.where(a16 == 1, N, 0)
            cps = [pltpu.async_copy(t_hbm.at[gis[b]], gbs[b], sem)
                   for b in range(_NB)]
            for b in range(_NB):
                cps[b].wait()
            for b in range(_NB):
                pltpu.sync_copy(gbs[b], stable.at[ris[b]], add=True)
            return 0

        lax.fori_loop(0, n_steps, step, 0)
        plsc.subcore_barrier()

        def cpout(kk, _):
            sl = pl.ds(s * slab + kk * _CH, _CH)
            pltpu.sync_copy(stable.at[sl], out_hbm.at[c, sl])
            return 0

        lax.fori_loop(0, slab // _CH, cpout, 0)

    return k(src_i, dst_i, attr, T)


def _matmul_bias_kernel(x_ref, w_ref, b_ref, o_ref):
    o_ref[...] = (
        jnp.dot(x_ref[...], w_ref[...], preferred_element_type=jnp.float32)
        + b_ref[...]
    )


def _matmul_bias(x, w, b, block_rows=256):
    m, k = x.shape
    n = w.shape[1]
    assert m % block_rows == 0
    return pl.pallas_call(
        _matmul_bias_kernel,
        grid=(m // block_rows,),
        in_specs=[
            pl.BlockSpec((block_rows, k), lambda i: (i, 0)),
            pl.BlockSpec((k, n), lambda i: (0, 0)),
            pl.BlockSpec((1, n), lambda i: (0, 0)),
        ],
        out_specs=pl.BlockSpec((block_rows, n), lambda i: (i, 0)),
        out_shape=jax.ShapeDtypeStruct((m, n), jnp.float32),
    )(x, w, b.reshape(1, n))


def kernel(x_visual, x_audio, edge_index, edge_attr, speakers, W011, b011, W012, b012, W1_v11, b1_v11, W2_v11, b2_v11, W1_v12, b1_v12, W2_v12, b2_v12, W1_v13, b1_v13, W2_v13, b2_v13, W1_a13, b1_a13, W2_a13, b2_a13, Wl, bl, Wr, Wfa, bfa):
    NV = x_visual.shape[0] * x_visual.shape[1]
    C = W011.shape[1]

    xv = _matmul_bias(x_visual.reshape(-1, x_visual.shape[-1]), W011, b011)
    xa = _matmul_bias(x_audio.reshape(-1, x_audio.shape[-1]), W012, b012)
    x = jnp.concatenate([xv, xa], axis=0)
    N = x.shape[0]

    src, dst = edge_index[0], edge_index[1]
    attr = edge_attr

    w_scalar = bfa[0]
    X1 = jnp.concatenate(
        [x, jnp.ones((N, 1), jnp.float32), jnp.zeros((N, C - 1), jnp.float32)], 1)
    X1 = jnp.concatenate([X1, jnp.zeros((8, 2 * C), jnp.float32)], 0)
    idxg = jnp.where(attr == 3, src, N).astype(jnp.int32)
    pay = _gather_rows(idxg, X1)
    S = jnp.zeros((N, C + 1), jnp.float32).at[dst].add(pay[:, : C + 1])
    x_vis = jax.nn.relu(w_scalar * S[:, :C] + S[:, C:] * x)

    def split_w1(W1):
        return W1[:C] - W1[C:], W1[C:]

    Wd1, Ws1 = split_w1(W1_v11)
    Wd2, Ws2 = split_w1(W1_v12)
    Wd3, Ws3 = split_w1(W1_v13)
    zc = jnp.zeros((C, C), jnp.float32)
    Wd = jnp.concatenate([Wd1, Wd3, Wd2, zc], axis=1)
    Ws = jnp.concatenate([Ws1, Ws3, Ws2, zc], axis=1)
    b1 = jnp.concatenate([b1_v11, b1_v13, b1_v12, jnp.zeros((C,), jnp.float32)])
    A = _matmul_bias(x_vis, Wd, b1)
    B = _matmul_bias(x_vis, Ws, jnp.zeros((4 * C,), jnp.float32))

    h1 = jax.nn.relu(_gather_pair_add(dst, src, A, B, 4 * C))
    W2 = jnp.stack([W2_v11, W2_v13, W2_v12, zc])
    b2 = jnp.stack([b2_v11, b2_v13, b2_v12, jnp.zeros((C,), jnp.float32)])
    msg = jax.nn.relu(
        jnp.einsum("ebc,bcd->ebd", h1.reshape(-1, 4, C), W2,
                   preferred_element_type=jnp.float32) + b2).reshape(-1, 4 * C)

    mask13 = attr <= 1
    mask2 = attr == 0
    dst13 = jnp.where(mask13, dst, N).astype(jnp.int32)
    dst2 = jnp.where(mask2, dst, N).astype(jnp.int32)
    H13 = jnp.zeros((N + 1, 2 * C), jnp.float32).at[dst13].max(msg[:, : 2 * C])
    H2f = jnp.zeros((N + 1, C), jnp.float32).at[dst2].max(msg[:, 2 * C : 3 * C])

    H1, H3, H2 = H13[:N, :C], H13[:N, C:], H2f[:N]
    zN = jnp.zeros((N, C), jnp.float32)
    oN = jnp.ones((N, 1), jnp.float32)
    z62 = jnp.zeros((N, C - 2), jnp.float32)
    T = jnp.concatenate([
        jnp.concatenate([H1, H2], 1),
        jnp.concatenate([H1, zN], 1),
        jnp.concatenate([H3, oN, oN, z62], 1),
        jnp.concatenate([H3, oN, jnp.zeros((N, 1), jnp.float32), z62], 1),
    ], 0)
    agg2 = _sage_agg(src, dst, attr, T)
    lo, hi = agg2[0, :9600], agg2[1, :9600]

    def take_main(arr):
        return arr[:NV].reshape(150, 4, 64, arr.shape[-1])[:, 0].reshape(9600, arr.shape[-1])

    cnt13 = jnp.maximum(hi[:, C], 1.0)[:, None]
    cnt2 = jnp.maximum(hi[:, C + 1], 1.0)[:, None]
    mean1 = lo[:, :C] / cnt13
    mean2 = lo[:, C:] / cnt2
    mean3 = hi[:, :C] / cnt13

    o1 = jax.nn.relu(mean1 @ Wl + bl + take_main(H1) @ Wr)
    o2 = jax.nn.relu(mean2 @ Wl + bl + take_main(H2) @ Wr)
    o3 = jax.nn.relu(mean3 @ Wl + bl + take_main(H3) @ Wr)
    return o1 + o2 + o3


# speedup vs baseline: 2.9308x; 2.9308x over previous
import functools

import jax
import jax.numpy as jnp
import numpy as np
from jax import lax
from jax.experimental import pallas as pl
from jax.experimental.pallas import tpu as pltpu
from jax.experimental.pallas import tpu_sc as plsc

_NC, _NS, _LANES = 2, 16, 16
_NW = _NC * _NS
_CH = 128


def _gather_pair_add(dst_i, src_i, A, B, width):
    E = dst_i.shape[0]
    per_w = E // _NW
    n_chunks = per_w // _CH
    nvec = width // _LANES
    mesh = plsc.VectorSubcoreMesh(core_axis_name="c", subcore_axis_name="s")

    @functools.partial(
        pl.kernel,
        mesh=mesh,
        out_type=jax.ShapeDtypeStruct((E, width), jnp.float32),
        scratch_types=[
            pltpu.VMEM((_CH,), jnp.int32),
            pltpu.VMEM((_CH,), jnp.int32),
            pltpu.VMEM((_CH, width), jnp.float32),
            pltpu.VMEM((_CH, width), jnp.float32),
            pltpu.SemaphoreType.DMA,
            pltpu.SemaphoreType.DMA,
        ],
    )
    def k(dst_hbm, src_hbm, a_hbm, b_hbm, out_hbm, idx_d, idx_s, arows, brows, sem_a, sem_b):
        wid = lax.axis_index("s") * _NC + lax.axis_index("c")
        base = wid * per_w

        def chunk(ci, _):
            off = base + ci * _CH
            pltpu.sync_copy(dst_hbm.at[pl.ds(off, _CH)], idx_d)
            pltpu.sync_copy(src_hbm.at[pl.ds(off, _CH)], idx_s)
            ca = pltpu.async_copy(a_hbm.at[idx_d], arows, sem_a)
            cb = pltpu.async_copy(b_hbm.at[idx_s], brows, sem_b)
            ca.wait()
            cb.wait()

            def edge(e, _):
                for j in range(nvec):
                    sl = pl.ds(j * _LANES, _LANES)
                    arows[e, sl] = arows[e, sl] + brows[e, sl]
                return 0

            lax.fori_loop(0, _CH, edge, 0, unroll=2)
            pltpu.sync_copy(arows, out_hbm.at[pl.ds(off, _CH)])
            return 0

        lax.fori_loop(0, n_chunks, chunk, 0)

    return k(dst_i, src_i, A, B)


def _gather_rows(idx, table):
    E = idx.shape[0]
    per_w = E // _NW
    n_chunks = per_w // _CH
    mesh = plsc.VectorSubcoreMesh(core_axis_name="c", subcore_axis_name="s")

    @functools.partial(
        pl.kernel,
        mesh=mesh,
        out_type=jax.ShapeDtypeStruct((E, 128), jnp.float32),
        scratch_types=[
            pltpu.VMEM((_CH,), jnp.int32),
            pltpu.VMEM((_CH,), jnp.int32),
            pltpu.VMEM((_CH, 128), jnp.float32),
            pltpu.VMEM((_CH, 128), jnp.float32),
            pltpu.SemaphoreType.DMA,
            pltpu.SemaphoreType.DMA,
        ],
    )
    def k(idx_hbm, t_hbm, out_hbm, iv0, iv1, b0, b1, s0, s1):
        wid = lax.axis_index("s") * _NC + lax.axis_index("c")
        base = wid * per_w

        def chunk(ci, _):
            off = base + ci * (2 * _CH)
            pltpu.sync_copy(idx_hbm.at[pl.ds(off, _CH)], iv0)
            pltpu.sync_copy(idx_hbm.at[pl.ds(off + _CH, _CH)], iv1)
            c0 = pltpu.async_copy(t_hbm.at[iv0], b0, s0)
            c1 = pltpu.async_copy(t_hbm.at[iv1], b1, s1)
            c0.wait()
            pltpu.sync_copy(b0, out_hbm.at[pl.ds(off, _CH)])
            c1.wait()
            pltpu.sync_copy(b1, out_hbm.at[pl.ds(off + _CH, _CH)])
            return 0

        lax.fori_loop(0, n_chunks // 2, chunk, 0)

    return k(idx, table)


_NB = 2
_SROWS = 10240


def _sage_agg(src_i, dst_i, attr, T):
    E = src_i.shape[0]
    N = T.shape[0] // 4
    per_tile = E // _NS
    n_steps = per_tile // (_NB * _CH)
    mesh = plsc.VectorSubcoreMesh(core_axis_name="c", subcore_axis_name="s")

    @functools.partial(
        pl.kernel,
        mesh=mesh,
        out_type=jax.ShapeDtypeStruct((2, _SROWS, 128), jnp.float32),
        scratch_types=(
            [pltpu.VMEM((_NB * _CH,), jnp.int32)] * 3
            + [pltpu.VMEM((_CH,), jnp.int32)] * (2 * _NB)
            + [pltpu.VMEM((_CH, 128), jnp.float32)] * _NB
            + [pltpu.VMEM_SHARED((_SROWS, 128), jnp.float32),
               pltpu.SemaphoreType.DMA]
        ),
    )
    def k(src_hbm, dst_hbm, attr_hbm, t_hbm, out_hbm,
          sv, dv, av, gi0, gi1, ri0, ri1,
          gb0, gb1, stable, sem):
        c = lax.axis_index("c")
        s = lax.axis_index("s")
        gis = [gi0, gi1]
        ris = [ri0, ri1]
        gbs = [gb0, gb1]

        def zrow(r, _):
            for j in range(8):
                gb0[r, pl.ds(j * _LANES, _LANES)] = jnp.zeros((_LANES,), jnp.float32)
            return 0

        lax.fori_loop(0, _CH, zrow, 0)
        slab = _SROWS // _NS

        def zslab(kk, _):
            stable_blk = stable.at[pl.ds(s * slab + kk * _CH, _CH)]
            pltpu.sync_copy(gb0, stable_blk)
            return 0

        lax.fori_loop(0, slab // _CH, zslab, 0)
        plsc.subcore_barrier()

        base = s * per_tile
        c2n = c * (2 * N)

        def step(i, _):
            off = base + i * (_NB * _CH)
            pltpu.sync_copy(src_hbm.at[pl.ds(off, _NB * _CH)], sv)
            pltpu.sync_copy(dst_hbm.at[pl.ds(off, _NB * _CH)], dv)
            pltpu.sync_copy(attr_hbm.at[pl.ds(off, _NB * _CH)], av)
            for b in range(_NB):
                for j in range(_CH // _LANES):
                    sl = pl.ds(b * _CH + j * _LANES, _LANES)
                    sl_o = pl.ds(j * _LANES, _LANES)
                    s16 = sv[sl]
                    d16 = dv[sl]
                    a16 = av[sl]
                    valid = ((a16 <= 1) & (d16 < 38400)
                             & ((d16 & 255) < 64))
                    srow = ((d16 >> 8) << 6) | (d16 & 63)
                    ris[b][sl_o] = jnp.where(valid, srow, 9600)
                    gis[b][sl_o] = c2n + s16 + jnp.where(a16 == 1, N, 0)
            cps = [pltpu.async_copy(t_hbm.at[gis[b]], gbs[b], sem)
                   for b in range(_NB)]
            for b in range(_NB):
                cps[b].wait()
            for b in range(_NB):
                pltpu.sync_copy(gbs[b], stable.at[ris[b]], add=True)
            return 0

        lax.fori_loop(0, n_steps, step, 0)
        plsc.subcore_barrier()

        def cpout(kk, _):
            sl = pl.ds(s * slab + kk * _CH, _CH)
            pltpu.sync_copy(stable.at[sl], out_hbm.at[c, sl])
            return 0

        lax.fori_loop(0, slab // _CH, cpout, 0)

    return k(src_i, dst_i, attr, T)


def _matmul_bias_kernel(x_ref, w_ref, b_ref, o_ref):
    o_ref[...] = (
        jnp.dot(x_ref[...], w_ref[...], preferred_element_type=jnp.float32)
        + b_ref[...]
    )


def _matmul_bias(x, w, b, block_rows=256):
    m, k = x.shape
    n = w.shape[1]
    assert m % block_rows == 0
    return pl.pallas_call(
        _matmul_bias_kernel,
        grid=(m // block_rows,),
        in_specs=[
            pl.BlockSpec((block_rows, k), lambda i: (i, 0)),
            pl.BlockSpec((k, n), lambda i: (0, 0)),
            pl.BlockSpec((1, n), lambda i: (0, 0)),
        ],
        out_specs=pl.BlockSpec((block_rows, n), lambda i: (i, 0)),
        out_shape=jax.ShapeDtypeStruct((m, n), jnp.float32),
    )(x, w, b.reshape(1, n))


def kernel(x_visual, x_audio, edge_index, edge_attr, speakers, W011, b011, W012, b012, W1_v11, b1_v11, W2_v11, b2_v11, W1_v12, b1_v12, W2_v12, b2_v12, W1_v13, b1_v13, W2_v13, b2_v13, W1_a13, b1_a13, W2_a13, b2_a13, Wl, bl, Wr, Wfa, bfa):
    NV = x_visual.shape[0] * x_visual.shape[1]
    C = W011.shape[1]

    xv = _matmul_bias(x_visual.reshape(-1, x_visual.shape[-1]), W011, b011)
    xa = _matmul_bias(x_audio.reshape(-1, x_audio.shape[-1]), W012, b012)
    x = jnp.concatenate([xv, xa], axis=0)
    N = x.shape[0]

    src, dst = edge_index[0], edge_index[1]
    attr = edge_attr

    w_scalar = bfa[0]
    X1 = jnp.concatenate(
        [x, jnp.ones((N, 1), jnp.float32), jnp.zeros((N, C - 1), jnp.float32)], 1)
    X1 = jnp.concatenate([X1, jnp.zeros((N, 2 * C), jnp.float32)], 0)
    idxg = (src + jnp.where(attr == 3, 0, N)).astype(jnp.int32)
    pay = _gather_rows(idxg, X1)
    S = jnp.zeros((N, C + 1), jnp.float32).at[dst].add(pay[:, : C + 1])
    x_vis = jax.nn.relu(w_scalar * S[:, :C] + S[:, C:] * x)

    def split_w1(W1):
        return W1[:C] - W1[C:], W1[C:]

    Wd1, Ws1 = split_w1(W1_v11)
    Wd2, Ws2 = split_w1(W1_v12)
    Wd3, Ws3 = split_w1(W1_v13)
    zc = jnp.zeros((C, C), jnp.float32)
    Wd = jnp.concatenate([Wd1, Wd3, Wd2, zc], axis=1)
    Ws = jnp.concatenate([Ws1, Ws3, Ws2, zc], axis=1)
    b1 = jnp.concatenate([b1_v11, b1_v13, b1_v12, jnp.zeros((C,), jnp.float32)])
    A = _matmul_bias(x_vis, Wd, b1)
    B = _matmul_bias(x_vis, Ws, jnp.zeros((4 * C,), jnp.float32))

    h1 = jax.nn.relu(_gather_pair_add(dst, src, A, B, 4 * C))
    W2 = jnp.stack([W2_v11, W2_v13, W2_v12, zc])
    b2 = jnp.stack([b2_v11, b2_v13, b2_v12, jnp.zeros((C,), jnp.float32)])
    msg = jax.nn.relu(
        jnp.einsum("ebc,bcd->ebd", h1.reshape(-1, 4, C), W2,
                   preferred_element_type=jnp.float32) + b2).reshape(-1, 4 * C)

    mask13 = attr <= 1
    mask2 = attr == 0
    dst13 = jnp.where(mask13, dst, N).astype(jnp.int32)
    pay192 = jnp.concatenate(
        [msg[:, : 2 * C],
         jnp.where(mask2[:, None], msg[:, 2 * C : 3 * C], 0.0)], 1)
    Hm = jnp.zeros((N + 1, 3 * C), jnp.float32).at[dst13].max(pay192)

    H1, H3, H2 = Hm[:N, :C], Hm[:N, C : 2 * C], Hm[:N, 2 * C :]
    zN = jnp.zeros((N, C), jnp.float32)
    oN = jnp.ones((N, 1), jnp.float32)
    z62 = jnp.zeros((N, C - 2), jnp.float32)
    T = jnp.concatenate([
        jnp.concatenate([H1, H2], 1),
        jnp.concatenate([H1, zN], 1),
        jnp.concatenate([H3, oN, oN, z62], 1),
        jnp.concatenate([H3, oN, jnp.zeros((N, 1), jnp.float32), z62], 1),
    ], 0)
    agg2 = _sage_agg(src, dst, attr, T)
    lo, hi = agg2[0, :9600], agg2[1, :9600]

    def take_main(arr):
        return arr[:NV].reshape(150, 4, 64, arr.shape[-1])[:, 0].reshape(9600, arr.shape[-1])

    cnt13 = jnp.maximum(hi[:, C], 1.0)[:, None]
    cnt2 = jnp.maximum(hi[:, C + 1], 1.0)[:, None]
    mean1 = lo[:, :C] / cnt13
    mean2 = lo[:, C:] / cnt2
    mean3 = hi[:, :C] / cnt13

    o1 = jax.nn.relu(mean1 @ Wl + bl + take_main(H1) @ Wr)
    o2 = jax.nn.relu(mean2 @ Wl + bl + take_main(H2) @ Wr)
    o3 = jax.nn.relu(mean3 @ Wl + bl + take_main(H3) @ Wr)
    return o1 + o2 + o3
